# baseline (speedup 1.0000x reference)
import functools

import jax
import jax.numpy as jnp
from jax import lax
from jax.experimental import pallas as pl
from jax.experimental.pallas import tpu as pltpu
from jax.experimental.pallas import tpu_sc as plsc

B = 16384
H1 = 128
NEG = 64
V = 20

NC = 2
NS = 16
L = 16
NW = NC * NS
BPW = B // NW
GROUPS = BPW // L


def _tables_body(w_ref, out_ref):
    w = w_ref[...]
    g = lax.dot_general(
        w, w,
        dimension_numbers=(((1,), (1,)), ((), ())),
        preferred_element_type=jnp.float32,
        precision=lax.Precision.HIGHEST,
    )
    out_ref[0] = g
    out_ref[1] = jax.nn.log_sigmoid(g)
    out_ref[2] = jax.nn.log_sigmoid(-g)


def _make_tables(w):
    return pl.pallas_call(
        _tables_body,
        out_shape=jax.ShapeDtypeStruct((3, V, V), jnp.float32),
    )(w)


def _sc_body(bu, bi, blab, bw, pu, pv, nv, pw,
             dbu, dbv, dbnv, dbw, e2u, e2v, e2lab, tbl,
             out,
             tbl_v, nv_v, dbnv_v,
             bu_v, bi_v, blab_v, bw_v, pu_v, pv_v, pw_v,
             dbu_v, dbv_v, dbw_v, e2u_v, e2v_v, e2lab_v, out_v, dma_sem):
    wid = lax.axis_index("s") * NC + lax.axis_index("c")
    base = wid * BPW

    copies = [pltpu.async_copy(tbl, tbl_v, dma_sem),
              pltpu.async_copy(nv.at[:, pl.ds(base, BPW)], nv_v, dma_sem),
              pltpu.async_copy(dbnv.at[:, pl.ds(base, BPW)], dbnv_v, dma_sem)]
    for hbm, vmem in ((bu, bu_v), (bi, bi_v), (blab, blab_v), (bw, bw_v),
                      (pu, pu_v), (pv, pv_v), (pw, pw_v),
                      (dbu, dbu_v), (dbv, dbv_v), (dbw, dbw_v),
                      (e2u, e2u_v), (e2v, e2v_v), (e2lab, e2lab_v)):
        copies.append(pltpu.async_copy(hbm.at[pl.ds(base, BPW)], vmem,
                                       dma_sem))
    for c in copies:
        c.wait()

    @plsc.parallel_loop(0, GROUPS, 1, unroll=3)
    def group(g):
        o = g * L
        bu16 = bu_v[pl.ds(o, L)]
        bi16 = bi_v[pl.ds(o, L)]
        pu16 = pu_v[pl.ds(o, L)]
        pv16 = pv_v[pl.ds(o, L)]
        dbu16 = dbu_v[pl.ds(o, L)]
        dbv16 = dbv_v[pl.ds(o, L)]
        e2u16 = e2u_v[pl.ds(o, L)]
        e2v16 = e2v_v[pl.ds(o, L)]

        pu20 = pu16 * V
        dbu20 = dbu16 * V
        rating = plsc.load_gather(tbl_v, [bu16 * V + bi16])
        pos_ls = plsc.load_gather(tbl_v, [pu20 + pv16 + V * V])
        db_pos_ls = plsc.load_gather(tbl_v, [dbu20 + dbv16 + V * V])
        e2 = plsc.load_gather(tbl_v, [e2u16 * V + e2v16])

        negbase = pu20 + 2 * V * V
        dbnegbase = dbu20 + 2 * V * V
        acc = [pos_ls, jnp.zeros((L,), jnp.float32)]
        dbacc = [db_pos_ls, jnp.zeros((L,), jnp.float32)]
        for n in range(NEG):
            k = n & 1
            nv16 = nv_v[n, pl.ds(o, L)]
            acc[k] = acc[k] + plsc.load_gather(tbl_v, [negbase + nv16])
            dbnv16 = dbnv_v[n, pl.ds(o, L)]
            dbacc[k] = dbacc[k] + plsc.load_gather(tbl_v, [dbnegbase + dbnv16])

        sg = -(acc[0] + acc[1]) * pw_v[pl.ds(o, L)]
        db_sg = -(dbacc[0] + dbacc[1]) * dbw_v[pl.ds(o, L)]
        res = (rating * bw_v[pl.ds(o, L)]
               + 0.1 * (sg + db_sg)
               + 0.01 * (e2 + e2lab_v[pl.ds(o, L)])
               + 0.001 * blab_v[pl.ds(o, L)])
        out_v[pl.ds(o, L)] = res

    pltpu.sync_copy(out_v, out.at[pl.ds(base, BPW)])


@functools.cache
def _make_sc_call():
    return pl.kernel(
        _sc_body,
        out_type=jax.ShapeDtypeStruct((B,), jnp.float32),
        mesh=plsc.VectorSubcoreMesh(core_axis_name="c", subcore_axis_name="s",
                                    num_cores=NC, num_subcores=NS),
        compiler_params=pltpu.CompilerParams(needs_layout_passes=False),
        scratch_types=[
        pltpu.VMEM((3 * V * V,), jnp.float32),
        pltpu.VMEM((NEG, BPW), jnp.int32),
        pltpu.VMEM((NEG, BPW), jnp.int32),
        pltpu.VMEM((BPW,), jnp.int32),
        pltpu.VMEM((BPW,), jnp.int32),
        pltpu.VMEM((BPW,), jnp.float32),
        pltpu.VMEM((BPW,), jnp.float32),
        pltpu.VMEM((BPW,), jnp.int32),
        pltpu.VMEM((BPW,), jnp.int32),
        pltpu.VMEM((BPW,), jnp.float32),
        pltpu.VMEM((BPW,), jnp.int32),
        pltpu.VMEM((BPW,), jnp.int32),
        pltpu.VMEM((BPW,), jnp.float32),
        pltpu.VMEM((BPW,), jnp.int32),
        pltpu.VMEM((BPW,), jnp.int32),
        pltpu.VMEM((BPW,), jnp.float32),
        pltpu.VMEM((BPW,), jnp.float32),
        pltpu.SemaphoreType.DMA,
        ],
    )


@jax.jit
def kernel(batch_uid, batch_iid, batch_label, batch_w, pos_u, pos_v, neg_v,
           pos_w, db_pos_u, db_pos_v, db_neg_v, db_pos_w,
           batch_e2_uid, batch_e2_iid, batch_e2_label, W_userDoc):
    tbl = _make_tables(W_userDoc).reshape(3 * V * V)
    return _make_sc_call()(batch_uid, batch_iid, batch_label, batch_w,
                    pos_u, pos_v, jnp.swapaxes(neg_v, 0, 1), pos_w,
                    db_pos_u, db_pos_v, jnp.swapaxes(db_neg_v, 0, 1), db_pos_w,
                    batch_e2_uid, batch_e2_iid, batch_e2_label, tbl)


# speedup vs baseline: 1.0467x; 1.0467x over previous
import functools

import jax
import jax.numpy as jnp
from jax import lax
from jax.experimental import pallas as pl
from jax.experimental.pallas import tpu as pltpu
from jax.experimental.pallas import tpu_sc as plsc

B = 16384
H1 = 128
NEG = 64
V = 20

NC = 2
NS = 16
L = 16
NW = NC * NS
BPW = B // NW
GROUPS = BPW // L


def _tables_body(w_ref, out_ref):
    w = w_ref[...]
    g = lax.dot_general(
        w, w,
        dimension_numbers=(((1,), (1,)), ((), ())),
        preferred_element_type=jnp.float32,
        precision=lax.Precision.HIGHEST,
    )
    out_ref[0] = g
    out_ref[1] = jax.nn.log_sigmoid(g)
    out_ref[2] = jax.nn.log_sigmoid(-g)


def _make_tables(w):
    return pl.pallas_call(
        _tables_body,
        out_shape=jax.ShapeDtypeStruct((3, V, V), jnp.float32),
    )(w)


def _sc_body(bu, bi, blab, bw, pu, pv, nv, pw,
             dbu, dbv, dbnv, dbw, e2u, e2v, e2lab, tbl,
             out,
             tbl_v, nv_v, dbnv_v,
             bu_v, bi_v, blab_v, bw_v, pu_v, pv_v, pw_v,
             dbu_v, dbv_v, dbw_v, e2u_v, e2v_v, e2lab_v, out_v, dma_sem):
    wid = lax.axis_index("s") * NC + lax.axis_index("c")
    base = wid * BPW

    copies = [pltpu.async_copy(tbl, tbl_v, dma_sem),
              pltpu.async_copy(nv.at[:, pl.ds(base, BPW)], nv_v, dma_sem),
              pltpu.async_copy(dbnv.at[:, pl.ds(base, BPW)], dbnv_v, dma_sem)]
    for hbm, vmem in ((bu, bu_v), (bi, bi_v), (blab, blab_v), (bw, bw_v),
                      (pu, pu_v), (pv, pv_v), (pw, pw_v),
                      (dbu, dbu_v), (dbv, dbv_v), (dbw, dbw_v),
                      (e2u, e2u_v), (e2v, e2v_v), (e2lab, e2lab_v)):
        copies.append(pltpu.async_copy(hbm.at[pl.ds(base, BPW)], vmem,
                                       dma_sem))
    for c in copies:
        c.wait()

    @plsc.parallel_loop(0, GROUPS, 1, unroll=2)
    def group(g):
        o = g * L
        bu16 = bu_v[pl.ds(o, L)]
        bi16 = bi_v[pl.ds(o, L)]
        pu16 = pu_v[pl.ds(o, L)]
        pv16 = pv_v[pl.ds(o, L)]
        dbu16 = dbu_v[pl.ds(o, L)]
        dbv16 = dbv_v[pl.ds(o, L)]
        e2u16 = e2u_v[pl.ds(o, L)]
        e2v16 = e2v_v[pl.ds(o, L)]

        pu20 = pu16 * V
        dbu20 = dbu16 * V
        rating = plsc.load_gather(tbl_v, [bu16 * V + bi16])
        pos_ls = plsc.load_gather(tbl_v, [pu20 + pv16 + V * V])
        db_pos_ls = plsc.load_gather(tbl_v, [dbu20 + dbv16 + V * V])
        e2 = plsc.load_gather(tbl_v, [e2u16 * V + e2v16])

        negbase = pu20 + 2 * V * V
        dbnegbase = dbu20 + 2 * V * V
        acc = [pos_ls, jnp.zeros((L,), jnp.float32)]
        dbacc = [db_pos_ls, jnp.zeros((L,), jnp.float32)]
        for n in range(NEG):
            k = n & 1
            nv16 = nv_v[n, pl.ds(o, L)]
            acc[k] = acc[k] + plsc.load_gather(tbl_v, [negbase + nv16])
            dbnv16 = dbnv_v[n, pl.ds(o, L)]
            dbacc[k] = dbacc[k] + plsc.load_gather(tbl_v, [dbnegbase + dbnv16])

        sg = -(acc[0] + acc[1]) * pw_v[pl.ds(o, L)]
        db_sg = -(dbacc[0] + dbacc[1]) * dbw_v[pl.ds(o, L)]
        res = (rating * bw_v[pl.ds(o, L)]
               + 0.1 * (sg + db_sg)
               + 0.01 * (e2 + e2lab_v[pl.ds(o, L)])
               + 0.001 * blab_v[pl.ds(o, L)])
        out_v[pl.ds(o, L)] = res

    pltpu.sync_copy(out_v, out.at[pl.ds(base, BPW)])


@functools.cache
def _make_sc_call():
    return pl.kernel(
        _sc_body,
        out_type=jax.ShapeDtypeStruct((B,), jnp.float32),
        mesh=plsc.VectorSubcoreMesh(core_axis_name="c", subcore_axis_name="s",
                                    num_cores=NC, num_subcores=NS),
        compiler_params=pltpu.CompilerParams(needs_layout_passes=False),
        scratch_types=[
        pltpu.VMEM((3 * V * V,), jnp.float32),
        pltpu.VMEM((NEG, BPW), jnp.int32),
        pltpu.VMEM((NEG, BPW), jnp.int32),
        pltpu.VMEM((BPW,), jnp.int32),
        pltpu.VMEM((BPW,), jnp.int32),
        pltpu.VMEM((BPW,), jnp.float32),
        pltpu.VMEM((BPW,), jnp.float32),
        pltpu.VMEM((BPW,), jnp.int32),
        pltpu.VMEM((BPW,), jnp.int32),
        pltpu.VMEM((BPW,), jnp.float32),
        pltpu.VMEM((BPW,), jnp.int32),
        pltpu.VMEM((BPW,), jnp.int32),
        pltpu.VMEM((BPW,), jnp.float32),
        pltpu.VMEM((BPW,), jnp.int32),
        pltpu.VMEM((BPW,), jnp.int32),
        pltpu.VMEM((BPW,), jnp.float32),
        pltpu.VMEM((BPW,), jnp.float32),
        pltpu.SemaphoreType.DMA,
        ],
    )


@jax.jit
def kernel(batch_uid, batch_iid, batch_label, batch_w, pos_u, pos_v, neg_v,
           pos_w, db_pos_u, db_pos_v, db_neg_v, db_pos_w,
           batch_e2_uid, batch_e2_iid, batch_e2_label, W_userDoc):
    tbl = _make_tables(W_userDoc).reshape(3 * V * V)
    return _make_sc_call()(batch_uid, batch_iid, batch_label, batch_w,
                    pos_u, pos_v, jnp.swapaxes(neg_v, 0, 1), pos_w,
                    db_pos_u, db_pos_v, jnp.swapaxes(db_neg_v, 0, 1), db_pos_w,
                    batch_e2_uid, batch_e2_iid, batch_e2_label, tbl)
